# baseline (speedup 1.0000x reference)
import functools

import jax
import jax.numpy as jnp
from jax import lax
from jax.experimental import pallas as pl
from jax.experimental.pallas import tpu as pltpu
from jax.experimental.pallas import tpu_sc as plsc

N = 10000
D = 128
NP = 10240
ROWS_PER_TILE = NP // 16
E = 320000
CHUNK = 128
CHUNKS_PER_TILE = 80
EDGES_PER_TILE = CHUNKS_PER_TILE * CHUNK
EP = EDGES_PER_TILE * 32
ROW_BLOCK = 640


def _linear_body(x_ref, w_ref, b_ref, c_ref, o_ref):
    acc = lax.dot_general(x_ref[...], w_ref[...], (((1,), (1,)), ((), ())),
                          preferred_element_type=jnp.float32)
    o_ref[...] = (acc + b_ref[...]) * c_ref[0]


def _linear(x, W, b, c):
    return pl.pallas_call(
        _linear_body,
        grid=(NP // ROW_BLOCK,),
        in_specs=[
            pl.BlockSpec((ROW_BLOCK, D), lambda i: (i, 0)),
            pl.BlockSpec((D, D), lambda i: (0, 0)),
            pl.BlockSpec((1, D), lambda i: (0, 0)),
            pl.BlockSpec(memory_space=pltpu.SMEM),
        ],
        out_specs=pl.BlockSpec((ROW_BLOCK, D), lambda i: (i, 0)),
        out_shape=jax.ShapeDtypeStruct((NP, D), jnp.float32),
    )(x, W, b, c)


def _mid_body(p_ref, w_ref, b_ref, c_ref, o_ref):
    h = jnp.maximum(p_ref[0] + p_ref[1], 0.0)
    acc = lax.dot_general(h, w_ref[...], (((1,), (1,)), ((), ())),
                          preferred_element_type=jnp.float32)
    o_ref[...] = (acc + b_ref[...]) * c_ref[0]


def _mid(p, W, b, c):
    return pl.pallas_call(
        _mid_body,
        grid=(NP // ROW_BLOCK,),
        in_specs=[
            pl.BlockSpec((2, ROW_BLOCK, D), lambda i: (0, i, 0)),
            pl.BlockSpec((D, D), lambda i: (0, 0)),
            pl.BlockSpec((1, D), lambda i: (0, 0)),
            pl.BlockSpec(memory_space=pltpu.SMEM),
        ],
        out_specs=pl.BlockSpec((ROW_BLOCK, D), lambda i: (i, 0)),
        out_shape=jax.ShapeDtypeStruct((NP, D), jnp.float32),
    )(p, W, b, c)


def _final_body(q_ref, o_ref):
    o = q_ref[0] + q_ref[1]
    m = jnp.max(o, axis=1, keepdims=True)
    e = jnp.exp(o - m)
    s = jnp.sum(e, axis=1, keepdims=True)
    o_ref[...] = o - m - jnp.log(s)


def _final(q):
    return pl.pallas_call(
        _final_body,
        grid=(N // 2000,),
        in_specs=[pl.BlockSpec((2, 2000, D), lambda i: (0, i, 0))],
        out_specs=pl.BlockSpec((2000, D), lambda i: (i, 0)),
        out_shape=jax.ShapeDtypeStruct((N, D), jnp.float32),
    )(q)


_SC_MESH = plsc.VectorSubcoreMesh(core_axis_name="c", subcore_axis_name="s")


CH0 = 104
CH1 = 160 - CH0


@functools.partial(
    pl.kernel,
    mesh=_SC_MESH,
    out_type=jax.ShapeDtypeStruct((2, NP, D), jnp.float32),
    scratch_types=[
        pltpu.VMEM((CH0 * CHUNK,), jnp.int32),
        pltpu.VMEM((CH0, CHUNK), jnp.int32),
        pltpu.VMEM((CHUNK, D), jnp.float32),
        pltpu.VMEM_SHARED((NP, D), jnp.float32),
        pltpu.SemaphoreType.DMA,
    ],
)
def _sc_scatter(y_hbm, z_hbm, src_hbm, dst2d_hbm, out_hbm,
                sidx, didx, rows, accum, sem):
    c = lax.axis_index("c")
    s = lax.axis_index("s")
    r0 = s * ROWS_PER_TILE
    ch_base = jnp.where(c == 1, s * CH1, 16 * CH1 + s * CH0)
    nch = jnp.where(c == 1, CH1, CH0)

    @pl.when(c == 0)
    def _():
        pltpu.sync_copy(y_hbm.at[pl.ds(r0, ROWS_PER_TILE)],
                        accum.at[pl.ds(r0, ROWS_PER_TILE)])

    @pl.when(c == 1)
    def _():
        pltpu.sync_copy(z_hbm.at[pl.ds(r0, ROWS_PER_TILE)],
                        accum.at[pl.ds(r0, ROWS_PER_TILE)])

    plsc.subcore_barrier()

    pltpu.sync_copy(src_hbm.at[pl.ds(ch_base * CHUNK, CH0 * CHUNK)], sidx)
    pltpu.sync_copy(dst2d_hbm.at[pl.ds(ch_base, CH0)], didx)

    def step(k, carry):
        pltpu.async_copy(y_hbm.at[sidx.at[pl.ds(k * CHUNK, CHUNK)]],
                         rows, sem).wait()
        pltpu.sync_copy(rows, accum.at[didx.at[k]], add=True)
        return carry

    lax.fori_loop(0, nch, step, 0)

    plsc.subcore_barrier()
    pltpu.sync_copy(accum.at[pl.ds(r0, ROWS_PER_TILE)],
                    out_hbm.at[c].at[pl.ds(r0, ROWS_PER_TILE)])


def kernel(x, edge_index, W1, b1, Wa1, ba1, c1, W2, b2, Wa2, ba2, c2):
    src = edge_index[0]
    dst = edge_index[1]
    pad_e = EP - E
    src_p = jnp.concatenate([src, jnp.zeros((pad_e,), src.dtype)])
    dst_p = jnp.concatenate([dst, jnp.full((pad_e,), N, dst.dtype)])
    dst2d = dst_p.reshape(CHUNKS_PER_TILE * 32, CHUNK)

    x_p = jnp.pad(x, ((0, NP - N), (0, 0)))
    z = jnp.zeros((NP, D), jnp.float32)
    c1s = -jnp.abs(c1)
    c2s = -jnp.abs(c2)

    y1 = _linear(x_p, W1, b1.reshape(1, D), c1s)
    p = _sc_scatter(y1, z, src_p, dst2d)
    y2 = _mid(p, W2, b2.reshape(1, D), c2s)
    q = _sc_scatter(y2, z, src_p, dst2d)
    return _final(q)


# speedup vs baseline: 1.1398x; 1.1398x over previous
import functools

import jax
import jax.numpy as jnp
from jax import lax
from jax.experimental import pallas as pl
from jax.experimental.pallas import tpu as pltpu
from jax.experimental.pallas import tpu_sc as plsc

N = 10000
D = 128
NP = 10240
ROWS_PER_TILE = NP // 16
E = 320000
CHUNK = 128
CHUNKS_PER_TILE = 80
EDGES_PER_TILE = CHUNKS_PER_TILE * CHUNK
EP = EDGES_PER_TILE * 32
ROW_BLOCK = 640


def _linear_body(x_ref, w_ref, b_ref, c_ref, o_ref):
    acc = lax.dot_general(x_ref[...], w_ref[...], (((1,), (1,)), ((), ())),
                          preferred_element_type=jnp.float32)
    o_ref[...] = (acc + b_ref[...]) * c_ref[0]


def _linear(x, W, b, c):
    return pl.pallas_call(
        _linear_body,
        grid=(NP // ROW_BLOCK,),
        in_specs=[
            pl.BlockSpec((ROW_BLOCK, D), lambda i: (i, 0)),
            pl.BlockSpec((D, D), lambda i: (0, 0)),
            pl.BlockSpec((1, D), lambda i: (0, 0)),
            pl.BlockSpec(memory_space=pltpu.SMEM),
        ],
        out_specs=pl.BlockSpec((ROW_BLOCK, D), lambda i: (i, 0)),
        out_shape=jax.ShapeDtypeStruct((NP, D), jnp.float32),
    )(x, W, b, c)


def _mid_body(p_ref, w_ref, b_ref, c_ref, o_ref):
    h = jnp.maximum(p_ref[0] + p_ref[1], 0.0)
    acc = lax.dot_general(h, w_ref[...], (((1,), (1,)), ((), ())),
                          preferred_element_type=jnp.float32)
    o_ref[...] = (acc + b_ref[...]) * c_ref[0]


def _mid(p, W, b, c):
    return pl.pallas_call(
        _mid_body,
        grid=(NP // ROW_BLOCK,),
        in_specs=[
            pl.BlockSpec((2, ROW_BLOCK, D), lambda i: (0, i, 0)),
            pl.BlockSpec((D, D), lambda i: (0, 0)),
            pl.BlockSpec((1, D), lambda i: (0, 0)),
            pl.BlockSpec(memory_space=pltpu.SMEM),
        ],
        out_specs=pl.BlockSpec((ROW_BLOCK, D), lambda i: (i, 0)),
        out_shape=jax.ShapeDtypeStruct((NP, D), jnp.float32),
    )(p, W, b, c)


def _final_body(q_ref, o_ref):
    o = q_ref[0] + q_ref[1]
    m = jnp.max(o, axis=1, keepdims=True)
    e = jnp.exp(o - m)
    s = jnp.sum(e, axis=1, keepdims=True)
    o_ref[...] = o - m - jnp.log(s)


def _final(q):
    return pl.pallas_call(
        _final_body,
        grid=(N // 2000,),
        in_specs=[pl.BlockSpec((2, 2000, D), lambda i: (0, i, 0))],
        out_specs=pl.BlockSpec((2000, D), lambda i: (i, 0)),
        out_shape=jax.ShapeDtypeStruct((N, D), jnp.float32),
    )(q)


_SC_MESH = plsc.VectorSubcoreMesh(core_axis_name="c", subcore_axis_name="s")


@functools.partial(
    pl.kernel,
    mesh=_SC_MESH,
    out_type=jax.ShapeDtypeStruct((2, NP, D), jnp.float32),
    scratch_types=[
        pltpu.VMEM((EDGES_PER_TILE,), jnp.int32),
        pltpu.VMEM((CHUNKS_PER_TILE, CHUNK), jnp.int32),
        pltpu.VMEM((CHUNK, D), jnp.float32),
        pltpu.VMEM_SHARED((NP, D), jnp.float32),
        pltpu.SemaphoreType.DMA,
    ],
)
def _sc_scatter(y_hbm, z_hbm, src_hbm, dst2d_hbm, out_hbm,
                sidx, didx, rows, accum, sem):
    c = lax.axis_index("c")
    s = lax.axis_index("s")
    wid = s * 2 + c
    r0 = s * ROWS_PER_TILE

    @pl.when(c == 0)
    def _():
        pltpu.sync_copy(y_hbm.at[pl.ds(r0, ROWS_PER_TILE)],
                        accum.at[pl.ds(r0, ROWS_PER_TILE)])

    @pl.when(c == 1)
    def _():
        pltpu.sync_copy(z_hbm.at[pl.ds(r0, ROWS_PER_TILE)],
                        accum.at[pl.ds(r0, ROWS_PER_TILE)])

    plsc.subcore_barrier()

    pltpu.sync_copy(src_hbm.at[pl.ds(wid * EDGES_PER_TILE, EDGES_PER_TILE)],
                    sidx)
    pltpu.sync_copy(dst2d_hbm.at[pl.ds(wid * CHUNKS_PER_TILE, CHUNKS_PER_TILE)],
                    didx)

    def step(k, carry):
        pltpu.async_copy(y_hbm.at[sidx.at[pl.ds(k * CHUNK, CHUNK)]],
                         rows, sem).wait()
        pltpu.sync_copy(rows, accum.at[didx.at[k]], add=True)
        return carry

    lax.fori_loop(0, CHUNKS_PER_TILE, step, 0)

    plsc.subcore_barrier()
    pltpu.sync_copy(accum.at[pl.ds(r0, ROWS_PER_TILE)],
                    out_hbm.at[c].at[pl.ds(r0, ROWS_PER_TILE)])


def kernel(x, edge_index, W1, b1, Wa1, ba1, c1, W2, b2, Wa2, ba2, c2):
    src = edge_index[0]
    dst = edge_index[1]
    pad_e = EP - E
    src_p = jnp.concatenate([src, jnp.zeros((pad_e,), src.dtype)])
    dst_p = jnp.concatenate([dst, jnp.full((pad_e,), N, dst.dtype)])
    dst2d = dst_p.reshape(CHUNKS_PER_TILE * 32, CHUNK)

    x_p = jnp.pad(x, ((0, NP - N), (0, 0)))
    z = jnp.zeros((NP, D), jnp.float32)
    c1s = -jnp.abs(c1)
    c2s = -jnp.abs(c2)

    y1 = _linear(x_p, W1, b1.reshape(1, D), c1s)
    p = _sc_scatter(y1, z, src_p, dst2d)
    y2 = _mid(p, W2, b2.reshape(1, D), c2s)
    q = _sc_scatter(y2, z, src_p, dst2d)
    return _final(q)
